# baseline (speedup 1.0000x reference)
import functools

import jax
import jax.numpy as jnp
from jax import lax
from jax.experimental import pallas as pl
from jax.experimental.pallas import tpu as pltpu
from jax.experimental.pallas import tpu_sc as plsc

N_VOCAB = 2048
SEQ = 8192
SEQ_OUT = 8193
NC, NS = 2, 16
NW = NC * NS
W_PER_B = NW // 4
ROWS_PER_W = SEQ // W_PER_B
G = 16
NGROUPS = ROWS_PER_W // G

_mesh = plsc.VectorSubcoreMesh(
    core_axis_name="c", subcore_axis_name="s", num_cores=NC, num_subcores=NS
)


@functools.partial(
    pl.kernel,
    out_type=jax.ShapeDtypeStruct((4, SEQ_OUT, N_VOCAB), jnp.float32),
    mesh=_mesh,
    scratch_types=[
        pltpu.VMEM((ROWS_PER_W,), jnp.int32),
        pltpu.VMEM((G, N_VOCAB), jnp.float32),
        pltpu.VMEM((G, N_VOCAB), jnp.float32),
        pltpu.VMEM((16,), jnp.int32),
        pltpu.SemaphoreType.DMA,
        pltpu.SemaphoreType.DMA,
    ],
    compiler_params=pltpu.CompilerParams(
        use_tc_tiling_on_sc=False, needs_layout_passes=False
    ),
)
def _sc_onehot(ids_hbm, zeros_hbm, tail_hbm, out_hbm,
               ids_v, buf0, buf1, tail_v, sem0, sem1):
    cid = lax.axis_index("c")
    sid = lax.axis_index("s")
    wid = sid * NC + cid
    b = wid // W_PER_B
    col0 = (wid % W_PER_B) * ROWS_PER_W

    rows16 = jnp.arange(G, dtype=jnp.int32)
    ones = jnp.full((G,), 1.0, jnp.float32)
    zeros16 = jnp.zeros((G,), jnp.float32)
    bufs = (buf0, buf1)
    sems = (sem0, sem1)

    pltpu.sync_copy(ids_hbm.at[b, pl.ds(col0, ROWS_PER_W)], ids_v)
    pltpu.sync_copy(zeros_hbm, buf0)
    pltpu.sync_copy(zeros_hbm, buf1)

    def put(buf, g, vals):
        idx = ids_v[pl.ds(g * G, G)]
        plsc.store_scatter(buf, [rows16, idx], vals)

    def start(buf, sem, g):
        pltpu.async_copy(buf, out_hbm.at[b, pl.ds(col0 + g * G, G), :], sem)

    def drain(buf, sem):
        pltpu.make_async_copy(buf, out_hbm.at[b, pl.ds(col0, G), :], sem).wait()

    put(buf0, 0, ones)
    start(buf0, sem0, 0)
    put(buf1, 1, ones)
    start(buf1, sem1, 1)

    def body(h, carry):
        for k in range(2):
            g = 2 * h + k
            drain(bufs[k], sems[k])
            put(bufs[k], g - 2, zeros16)
            put(bufs[k], g, ones)
            start(bufs[k], sems[k], g)
        return carry

    lax.fori_loop(1, NGROUPS // 2, body, 0)

    drain(buf0, sem0)
    drain(buf1, sem1)

    @pl.when(wid < 4)
    def _tail():
        put(buf0, NGROUPS - 2, zeros16)
        pltpu.sync_copy(tail_hbm, tail_v)
        plsc.store_scatter(
            buf0,
            [jnp.zeros((G,), jnp.int32), tail_v[...]],
            ones,
            mask=rows16 == wid,
        )
        pltpu.sync_copy(buf0.at[pl.ds(0, 1), :],
                        out_hbm.at[wid, pl.ds(SEQ, 1), :])


def kernel(input_ids):
    ids = input_ids.astype(jnp.int32)
    shifted = jnp.pad(ids, ((0, 0), (1, 0)))[:, :SEQ]
    tail = jnp.pad(ids[:, -1], (0, 12))
    zeros_blk = jnp.zeros((G, N_VOCAB), jnp.float32)
    return _sc_onehot(shifted, zeros_blk, tail)
